# baseline (speedup 1.0000x reference)
import functools

import jax
import jax.numpy as jnp
from jax import lax
from jax.experimental import pallas as pl
from jax.experimental.pallas import tpu as pltpu
from jax.experimental.pallas import tpu_sc as plsc

_DIM = 64
_NE = 1024
_NT = 16384
_TBLK = 512
_NB = _NT // _TBLK


def _tc_body(x_ref, embed_ref, cs_ref, ind_ref, loss_ref, nsmall_ref):
    i = pl.program_id(0)
    x = x_ref[...]
    e = embed_ref[...]
    x2 = jnp.sum(x * x, axis=1, keepdims=True)
    e2 = jnp.sum(e * e, axis=0, keepdims=True)
    xe = jnp.dot(x, e, preferred_element_type=jnp.float32)
    dist = x2 - 2.0 * xe + e2
    minval = jnp.min(dist, axis=1, keepdims=True)
    iota = lax.broadcasted_iota(jnp.int32, dist.shape, 1)
    ind = jnp.min(jnp.where(dist == minval, iota, _NE), axis=1)
    ind_ref[...] = ind

    @pl.when(i == 0)
    def _init():
        loss_ref[0, 0] = 0.0
        nsmall_ref[0, 0] = jnp.sum((cs_ref[...] < 1.0).astype(jnp.int32))

    s = loss_ref[0, 0] + jnp.sum(minval)
    loss_ref[0, 0] = jnp.where(i == _NB - 1, s * (1.0 / (_NT * _DIM)), s)


_tc_call = pl.pallas_call(
    _tc_body,
    grid=(_NB,),
    in_specs=[
        pl.BlockSpec((_TBLK, _DIM), lambda i: (i, 0)),
        pl.BlockSpec((_DIM, _NE), lambda i: (0, 0)),
        pl.BlockSpec((_NE,), lambda i: (0,)),
    ],
    out_specs=[
        pl.BlockSpec((_TBLK,), lambda i: (i,)),
        pl.BlockSpec((1, 1), lambda i: (0, 0), memory_space=pltpu.SMEM),
        pl.BlockSpec((1, 1), lambda i: (0, 0), memory_space=pltpu.SMEM),
    ],
    out_shape=[
        jax.ShapeDtypeStruct((_NT,), jnp.int32),
        jax.ShapeDtypeStruct((1, 1), jnp.float32),
        jax.ShapeDtypeStruct((1, 1), jnp.int32),
    ],
)


_NC, _NS, _L = 2, 16, 16
_NW = _NC * _NS
_BPW = _NT // _NW
_CHUNK = 128
_NCH = _BPW // _CHUNK


def _sc_gather_body(table_hbm, idx_hbm, out_hbm, idx_v, rows_v, sem):
    wid = lax.axis_index("s") * _NC + lax.axis_index("c")
    pltpu.sync_copy(idx_hbm.at[wid], idx_v)
    copies = [
        pltpu.async_copy(
            table_hbm.at[idx_v.at[j]],
            rows_v.at[pl.ds(j * _CHUNK, _CHUNK)],
            sem,
        )
        for j in range(_NCH)
    ]
    for c in copies:
        c.wait()
    pltpu.sync_copy(rows_v, out_hbm.at[pl.ds(wid * _BPW, _BPW)])


@functools.cache
def _sc_gather():
    return pl.kernel(
        _sc_gather_body,
        mesh=plsc.VectorSubcoreMesh(core_axis_name="c", subcore_axis_name="s"),
        out_type=jax.ShapeDtypeStruct((_NT, _DIM), jnp.float32),
        scratch_types=[
            pltpu.VMEM((_NCH, _CHUNK), jnp.int32),
            pltpu.VMEM((_BPW, _DIM), jnp.float32),
            pltpu.SemaphoreType.DMA,
        ],
        compiler_params=pltpu.CompilerParams(use_tc_tiling_on_sc=False),
    )


def kernel(x, embed, cluster_size):
    ind, loss, nsmall = _tc_call(x, embed, cluster_size)
    table = jnp.transpose(embed)
    idx3 = jnp.reshape(ind, (_NW, _NCH, _CHUNK))
    quantize = _sc_gather()(table, idx3)
    return (quantize, loss[0, 0], nsmall[0, 0], ind)


# speedup vs baseline: 1.1235x; 1.1235x over previous
import functools

import jax
import jax.numpy as jnp
from jax import lax
from jax.experimental import pallas as pl
from jax.experimental.pallas import tpu as pltpu
from jax.experimental.pallas import tpu_sc as plsc

_DIM = 64
_NE = 1024
_NT = 16384
_TBLK = 2048
_NB = _NT // _TBLK


def _tc_body(x_ref, embed_ref, cs_ref, ind_ref, loss_ref, nsmall_ref, e2_ref,
             iota_ref):
    i = pl.program_id(0)
    x = x_ref[...]
    e = embed_ref[...]

    @pl.when(i == 0)
    def _init():
        e2_ref[...] = jnp.sum(e * e, axis=0, keepdims=True)
        iota_ref[...] = lax.broadcasted_iota(
            jnp.int32, (1, _NE), 1).astype(jnp.float32)
        loss_ref[0, 0] = 0.0
        nsmall_ref[0, 0] = jnp.sum((cs_ref[...] < 1.0).astype(jnp.int32))

    x2 = jnp.sum(x * x, axis=1, keepdims=True)
    xe2 = jnp.dot(x * -2.0, e, preferred_element_type=jnp.float32)
    dist = (x2 + xe2) + e2_ref[...]
    minval = jnp.min(dist, axis=1, keepdims=True)
    iota = jnp.broadcast_to(iota_ref[...], dist.shape)
    ind_f = jnp.min(jnp.where(dist == minval, iota, float(_NE)), axis=1)
    ind_ref[...] = ind_f.astype(jnp.int32)

    s = loss_ref[0, 0] + jnp.sum(minval)
    loss_ref[0, 0] = jnp.where(i == _NB - 1, s * (1.0 / (_NT * _DIM)), s)


_tc_call = pl.pallas_call(
    _tc_body,
    grid=(_NB,),
    in_specs=[
        pl.BlockSpec((_TBLK, _DIM), lambda i: (i, 0)),
        pl.BlockSpec((_DIM, _NE), lambda i: (0, 0)),
        pl.BlockSpec((_NE,), lambda i: (0,)),
    ],
    out_specs=[
        pl.BlockSpec((_TBLK,), lambda i: (i,)),
        pl.BlockSpec((1, 1), lambda i: (0, 0), memory_space=pltpu.SMEM),
        pl.BlockSpec((1, 1), lambda i: (0, 0), memory_space=pltpu.SMEM),
    ],
    out_shape=[
        jax.ShapeDtypeStruct((_NT,), jnp.int32),
        jax.ShapeDtypeStruct((1, 1), jnp.float32),
        jax.ShapeDtypeStruct((1, 1), jnp.int32),
    ],
    scratch_shapes=[
        pltpu.VMEM((1, _NE), jnp.float32),
        pltpu.VMEM((1, _NE), jnp.float32),
    ],
)


_NC, _NS, _L = 2, 16, 16
_NW = _NC * _NS
_BPW = _NT // _NW
_CHUNK = 128
_NCH = _BPW // _CHUNK


def _sc_gather_body(table_hbm, idx_hbm, out_hbm, idx_v, rows_v, sem):
    wid = lax.axis_index("s") * _NC + lax.axis_index("c")
    pltpu.sync_copy(idx_hbm.at[wid], idx_v)
    copies = [
        pltpu.async_copy(
            table_hbm.at[idx_v.at[j]],
            rows_v.at[pl.ds(j * _CHUNK, _CHUNK)],
            sem,
        )
        for j in range(_NCH)
    ]
    for c in copies:
        c.wait()
    pltpu.sync_copy(rows_v, out_hbm.at[pl.ds(wid * _BPW, _BPW)])


@functools.cache
def _sc_gather():
    return pl.kernel(
        _sc_gather_body,
        mesh=plsc.VectorSubcoreMesh(core_axis_name="c", subcore_axis_name="s"),
        out_type=jax.ShapeDtypeStruct((_NT, _DIM), jnp.float32),
        scratch_types=[
            pltpu.VMEM((_NCH, _CHUNK), jnp.int32),
            pltpu.VMEM((_BPW, _DIM), jnp.float32),
            pltpu.SemaphoreType.DMA,
        ],
        compiler_params=pltpu.CompilerParams(use_tc_tiling_on_sc=False),
    )


def kernel(x, embed, cluster_size):
    ind, loss, nsmall = _tc_call(x, embed, cluster_size)
    table = jnp.transpose(embed)
    idx3 = jnp.reshape(ind, (_NW, _NCH, _CHUNK))
    quantize = _sc_gather()(table, idx3)
    return (quantize, loss[0, 0], nsmall[0, 0], ind)
